# baseline (speedup 1.0000x reference)
import functools

import numpy as np
import jax
import jax.numpy as jnp
from jax import lax
from jax.experimental import pallas as pl
from jax.experimental.pallas import tpu as pltpu
from jax.experimental.pallas import tpu_sc as plsc

_NUM_SAMPLE = 4096

_NUM_CORES = 2
_NUM_SUBCORES = 16
_NUM_WORKERS = _NUM_CORES * _NUM_SUBCORES
_CHUNK = 128


@functools.lru_cache(maxsize=None)
def _flat_sample_indices(B: int, N: int, C: int) -> np.ndarray:
    cpu = jax.local_devices(backend="cpu")[0]
    with jax.ensure_compile_time_eval(), jax.default_device(cpu):
        keys = jax.random.split(jax.random.key(42), B)
        idx = jax.vmap(lambda k: jax.random.permutation(k, N)[:_NUM_SAMPLE])(keys)
    idx = np.asarray(jax.device_get(idx)).astype(np.int64)
    rows = idx + (np.arange(B, dtype=np.int64) * N)[:, None]
    elems = rows.reshape(-1, 1) * C + np.arange(C, dtype=np.int64)
    return elems.reshape(-1).astype(np.int32)


@functools.lru_cache(maxsize=None)
def _build_flatten(B: int, N: int, C: int):
    w_per_b = _NUM_WORKERS // B
    assert w_per_b * B == _NUM_WORKERS and N % w_per_b == 0
    rows_w = N // w_per_b
    elems_w = rows_w * C
    ROWS = 200
    MACROS = ROWS * C // 48
    WRITE_EVERY = 25
    n_chunks = rows_w // ROWS
    span = WRITE_EVERY * ROWS * C
    assert n_chunks * ROWS == rows_w and n_chunks % WRITE_EVERY == 0
    assert MACROS * 48 == ROWS * C and span % 8 == 0
    mesh = plsc.VectorSubcoreMesh(core_axis_name="c", subcore_axis_name="s")

    @functools.partial(
        pl.kernel,
        out_type=jax.ShapeDtypeStruct((B * N * C,), jnp.float32),
        mesh=mesh,
        scratch_types=[
            pltpu.VMEM((7 * 16,), jnp.int32),
            pltpu.VMEM((2, ROWS, C), jnp.float32),
            pltpu.VMEM((span,), jnp.float32),
            pltpu.SemaphoreType.DMA,
            pltpu.SemaphoreType.DMA,
        ],
        compiler_params=pltpu.CompilerParams(needs_layout_passes=False),
    )
    def flatten_kernel(pts_hbm, aux_hbm, out_hbm, aux_v, vbuf, out_v, sem0, sem1):
        wid = lax.axis_index("s") * _NUM_CORES + lax.axis_index("c")
        b = wid // w_per_b
        h = wid % w_per_b
        row0 = h * rows_w
        obase = wid * elems_w

        pltpu.sync_copy(aux_hbm, aux_v)
        phases = [
            (aux_v[pl.ds(32 * p, 16)], aux_v[pl.ds(32 * p + 16, 16)])
            for p in range(3)
        ]
        zvec = aux_v[pl.ds(96, 16)]

        def fire(k, buf, sem):
            pltpu.async_copy(
                pts_hbm.at[b, pl.ds(row0 + k * ROWS, ROWS), :],
                vbuf.at[buf],
                sem,
            )

        def wait(buf, sem):
            pltpu.make_async_copy(
                pts_hbm.at[b, pl.ds(row0, ROWS), :], vbuf.at[buf], sem
            ).wait()

        fire(0, 0, sem0)

        def chunk_body(k, carry):
            par = lax.rem(k, 2)
            nxt_ok = k + 1 < n_chunks
            nxt_par = lax.rem(k + 1, 2)

            @pl.when(jnp.logical_and(nxt_ok, nxt_par == 0))
            def _():
                fire(k + 1, 0, sem0)

            @pl.when(jnp.logical_and(nxt_ok, nxt_par == 1))
            def _():
                fire(k + 1, 1, sem1)

            @pl.when(par == 0)
            def _():
                wait(0, sem0)

            @pl.when(par == 1)
            def _():
                wait(1, sem1)

            bvec = zvec + par
            vbase = lax.rem(k, WRITE_EVERY) * (ROWS * C)

            for m in range(MACROS):
                for p, (iv, jv) in enumerate(phases):
                    vals = plsc.load_gather(vbuf, [bvec, iv + 8 * m, jv])
                    out_v[pl.ds(vbase + m * 48 + p * 16, 16)] = vals

            @pl.when(lax.rem(k, WRITE_EVERY) == WRITE_EVERY - 1)
            def _():
                pltpu.sync_copy(
                    out_v,
                    out_hbm.at[pl.ds(obase + (k // WRITE_EVERY) * span, span)],
                )

            return carry

        lax.fori_loop(0, n_chunks, chunk_body, 0)

    return flatten_kernel


@functools.lru_cache(maxsize=None)
def _build_gather(E: int):
    assert E % (_NUM_WORKERS * _CHUNK) == 0
    per_w = E // _NUM_WORKERS
    n_chunks = per_w // _CHUNK
    mesh = plsc.VectorSubcoreMesh(core_axis_name="c", subcore_axis_name="s")

    @functools.partial(
        pl.kernel,
        out_type=jax.ShapeDtypeStruct((E,), jnp.float32),
        mesh=mesh,
        scratch_types=[
            pltpu.VMEM((per_w,), jnp.int32),
            pltpu.VMEM((per_w,), jnp.float32),
            pltpu.SemaphoreType.DMA,
        ],
    )
    def gather_kernel(table_hbm, idx_hbm, out_hbm, idx_v, vals_v, sem):
        wid = lax.axis_index("s") * _NUM_CORES + lax.axis_index("c")
        base = wid * per_w
        pltpu.sync_copy(idx_hbm.at[pl.ds(base, per_w)], idx_v)

        def issue(c, carry):
            off = c * _CHUNK
            pltpu.async_copy(
                table_hbm.at[idx_v.at[pl.ds(off, _CHUNK)]],
                vals_v.at[pl.ds(off, _CHUNK)],
                sem,
            )
            return carry

        lax.fori_loop(0, n_chunks, issue, 0)
        pltpu.make_async_copy(table_hbm.at[pl.ds(0, per_w)], vals_v, sem).wait()
        pltpu.sync_copy(vals_v, out_hbm.at[pl.ds(base, per_w)])

    return gather_kernel


@functools.lru_cache(maxsize=None)
def _flatten_aux(C: int) -> np.ndarray:
    q = np.arange(48, dtype=np.int32)
    iv, jv = q // C, q % C
    parts = []
    for p in range(3):
        parts += [iv[16 * p : 16 * (p + 1)], jv[16 * p : 16 * (p + 1)]]
    parts.append(np.zeros(16, np.int32))
    return np.concatenate(parts)


def kernel(points):
    B, N, C = points.shape
    flat_idx = jnp.asarray(_flat_sample_indices(B, N, C))
    table = _build_flatten(B, N, C)(points, jnp.asarray(_flatten_aux(C)))
    out = _build_gather(B * _NUM_SAMPLE * C)(table, flat_idx)
    return out.reshape(B, _NUM_SAMPLE, C)


# speedup vs baseline: 1.0811x; 1.0811x over previous
import functools

import numpy as np
import jax
import jax.numpy as jnp
from jax import lax
from jax.experimental import pallas as pl
from jax.experimental.pallas import tpu as pltpu
from jax.experimental.pallas import tpu_sc as plsc

_NUM_SAMPLE = 4096

_NUM_CORES = 2
_NUM_SUBCORES = 16
_NUM_WORKERS = _NUM_CORES * _NUM_SUBCORES
_CHUNK = 128


@functools.lru_cache(maxsize=None)
def _flat_sample_indices(B: int, N: int, C: int) -> np.ndarray:
    cpu = jax.local_devices(backend="cpu")[0]
    with jax.ensure_compile_time_eval(), jax.default_device(cpu):
        keys = jax.random.split(jax.random.key(42), B)
        idx = jax.vmap(lambda k: jax.random.permutation(k, N)[:_NUM_SAMPLE])(keys)
    idx = np.asarray(jax.device_get(idx)).astype(np.int64)
    rows = idx + (np.arange(B, dtype=np.int64) * N)[:, None]
    elems = rows.reshape(-1, 1) * C + np.arange(C, dtype=np.int64)
    return elems.reshape(-1).astype(np.int32)


@functools.lru_cache(maxsize=None)
def _build_flatten(B: int, N: int, C: int):
    w_per_b = _NUM_WORKERS // B
    assert w_per_b * B == _NUM_WORKERS and N % w_per_b == 0
    rows_w = N // w_per_b
    elems_w = rows_w * C
    ROWS = 200
    NBUF = 4
    MACROS = ROWS * C // 48
    WRITE_EVERY = 10
    n_chunks = rows_w // ROWS
    span = WRITE_EVERY * ROWS * C
    assert n_chunks * ROWS == rows_w and n_chunks % WRITE_EVERY == 0
    assert MACROS * 48 == ROWS * C and span % 8 == 0
    mesh = plsc.VectorSubcoreMesh(core_axis_name="c", subcore_axis_name="s")

    @functools.partial(
        pl.kernel,
        out_type=jax.ShapeDtypeStruct((B * N * C,), jnp.float32),
        mesh=mesh,
        scratch_types=[
            pltpu.VMEM((7 * 16,), jnp.int32),
            pltpu.VMEM((NBUF, ROWS, C), jnp.float32),
            pltpu.VMEM((span,), jnp.float32),
            pltpu.SemaphoreType.DMA,
            pltpu.SemaphoreType.DMA,
            pltpu.SemaphoreType.DMA,
            pltpu.SemaphoreType.DMA,
        ],
        compiler_params=pltpu.CompilerParams(needs_layout_passes=False),
    )
    def flatten_kernel(
        pts_hbm, aux_hbm, out_hbm, aux_v, vbuf, out_v, sem0, sem1, sem2, sem3
    ):
        wid = lax.axis_index("s") * _NUM_CORES + lax.axis_index("c")
        b = wid // w_per_b
        h = wid % w_per_b
        row0 = h * rows_w
        obase = wid * elems_w

        pltpu.sync_copy(aux_hbm, aux_v)
        phases = [
            (aux_v[pl.ds(32 * p, 16)], aux_v[pl.ds(32 * p + 16, 16)])
            for p in range(3)
        ]
        zvec = aux_v[pl.ds(96, 16)]

        def fire(k, buf, sem):
            pltpu.async_copy(
                pts_hbm.at[b, pl.ds(row0 + k * ROWS, ROWS), :],
                vbuf.at[buf],
                sem,
            )

        def wait(buf, sem):
            pltpu.make_async_copy(
                pts_hbm.at[b, pl.ds(row0, ROWS), :], vbuf.at[buf], sem
            ).wait()

        sems = (sem0, sem1, sem2, sem3)
        for j in range(NBUF - 1):
            fire(j, j, sems[j])

        def chunk_body(k, carry):
            par = lax.rem(k, NBUF)
            nxt = k + NBUF - 1
            nxt_par = lax.rem(nxt, NBUF)

            for j in range(NBUF):
                @pl.when(jnp.logical_and(nxt < n_chunks, nxt_par == j))
                def _(j=j):
                    fire(nxt, j, sems[j])

            for j in range(NBUF):
                @pl.when(par == j)
                def _(j=j):
                    wait(j, sems[j])

            bvec = zvec + par
            vbase = lax.rem(k, WRITE_EVERY) * (ROWS * C)

            for m in range(MACROS):
                for p, (iv, jv) in enumerate(phases):
                    vals = plsc.load_gather(vbuf, [bvec, iv + 8 * m, jv])
                    out_v[pl.ds(vbase + m * 48 + p * 16, 16)] = vals

            @pl.when(lax.rem(k, WRITE_EVERY) == WRITE_EVERY - 1)
            def _():
                pltpu.sync_copy(
                    out_v,
                    out_hbm.at[pl.ds(obase + (k // WRITE_EVERY) * span, span)],
                )

            return carry

        lax.fori_loop(0, n_chunks, chunk_body, 0)

    return flatten_kernel


@functools.lru_cache(maxsize=None)
def _build_gather(E: int):
    assert E % (_NUM_WORKERS * _CHUNK) == 0
    per_w = E // _NUM_WORKERS
    n_chunks = per_w // _CHUNK
    mesh = plsc.VectorSubcoreMesh(core_axis_name="c", subcore_axis_name="s")

    @functools.partial(
        pl.kernel,
        out_type=jax.ShapeDtypeStruct((E,), jnp.float32),
        mesh=mesh,
        scratch_types=[
            pltpu.VMEM((per_w,), jnp.int32),
            pltpu.VMEM((per_w,), jnp.float32),
            pltpu.SemaphoreType.DMA,
        ],
    )
    def gather_kernel(table_hbm, idx_hbm, out_hbm, idx_v, vals_v, sem):
        wid = lax.axis_index("s") * _NUM_CORES + lax.axis_index("c")
        base = wid * per_w
        pltpu.sync_copy(idx_hbm.at[pl.ds(base, per_w)], idx_v)

        def issue(c, carry):
            off = c * _CHUNK
            pltpu.async_copy(
                table_hbm.at[idx_v.at[pl.ds(off, _CHUNK)]],
                vals_v.at[pl.ds(off, _CHUNK)],
                sem,
            )
            return carry

        lax.fori_loop(0, n_chunks, issue, 0)
        pltpu.make_async_copy(table_hbm.at[pl.ds(0, per_w)], vals_v, sem).wait()
        pltpu.sync_copy(vals_v, out_hbm.at[pl.ds(base, per_w)])

    return gather_kernel


@functools.lru_cache(maxsize=None)
def _flatten_aux(C: int) -> np.ndarray:
    q = np.arange(48, dtype=np.int32)
    iv, jv = q // C, q % C
    parts = []
    for p in range(3):
        parts += [iv[16 * p : 16 * (p + 1)], jv[16 * p : 16 * (p + 1)]]
    parts.append(np.zeros(16, np.int32))
    return np.concatenate(parts)


def kernel(points):
    B, N, C = points.shape
    flat_idx = jnp.asarray(_flat_sample_indices(B, N, C))
    table = _build_flatten(B, N, C)(points, jnp.asarray(_flatten_aux(C)))
    out = _build_gather(B * _NUM_SAMPLE * C)(table, flat_idx)
    return out.reshape(B, _NUM_SAMPLE, C)


# speedup vs baseline: 1.0874x; 1.0059x over previous
import functools

import numpy as np
import jax
import jax.numpy as jnp
from jax import lax
from jax.experimental import pallas as pl
from jax.experimental.pallas import tpu as pltpu
from jax.experimental.pallas import tpu_sc as plsc

_NUM_SAMPLE = 4096

_NUM_CORES = 2
_NUM_SUBCORES = 16
_NUM_WORKERS = _NUM_CORES * _NUM_SUBCORES
_CHUNK = 128


@functools.lru_cache(maxsize=None)
def _flat_sample_indices(B: int, N: int, C: int) -> np.ndarray:
    cpu = jax.local_devices(backend="cpu")[0]
    with jax.ensure_compile_time_eval(), jax.default_device(cpu):
        keys = jax.random.split(jax.random.key(42), B)
        idx = jax.vmap(lambda k: jax.random.permutation(k, N)[:_NUM_SAMPLE])(keys)
    idx = np.asarray(jax.device_get(idx)).astype(np.int64)
    rows = idx + (np.arange(B, dtype=np.int64) * N)[:, None]
    elems = rows.reshape(-1, 1) * C + np.arange(C, dtype=np.int64)
    return elems.reshape(-1).astype(np.int32)


@functools.lru_cache(maxsize=None)
def _build_flatten(B: int, N: int, C: int):
    w_per_b = _NUM_WORKERS // B
    assert w_per_b * B == _NUM_WORKERS and N % w_per_b == 0
    rows_w = N // w_per_b
    elems_w = rows_w * C
    ROWS = 200
    NBUF = 4
    MACROS = ROWS * C // 48
    WRITE_EVERY = 10
    n_chunks = rows_w // ROWS
    span = WRITE_EVERY * ROWS * C
    assert n_chunks * ROWS == rows_w and n_chunks % WRITE_EVERY == 0
    assert MACROS * 48 == ROWS * C and span % 8 == 0
    mesh = plsc.VectorSubcoreMesh(core_axis_name="c", subcore_axis_name="s")

    @functools.partial(
        pl.kernel,
        out_type=jax.ShapeDtypeStruct((B * N * C,), jnp.float32),
        mesh=mesh,
        scratch_types=[
            pltpu.VMEM((7 * 16,), jnp.int32),
            pltpu.VMEM((NBUF, ROWS, C), jnp.float32),
            pltpu.VMEM((2 * span,), jnp.float32),
            pltpu.SemaphoreType.DMA,
            pltpu.SemaphoreType.DMA,
            pltpu.SemaphoreType.DMA,
            pltpu.SemaphoreType.DMA,
            pltpu.SemaphoreType.DMA,
            pltpu.SemaphoreType.DMA,
        ],
        compiler_params=pltpu.CompilerParams(needs_layout_passes=False),
    )
    def flatten_kernel(
        pts_hbm, aux_hbm, out_hbm, aux_v, vbuf, out_v,
        sem0, sem1, sem2, sem3, semo0, semo1,
    ):
        wid = lax.axis_index("s") * _NUM_CORES + lax.axis_index("c")
        b = wid // w_per_b
        h = wid % w_per_b
        row0 = h * rows_w
        obase = wid * elems_w

        pltpu.sync_copy(aux_hbm, aux_v)
        phases = [
            (aux_v[pl.ds(32 * p, 16)], aux_v[pl.ds(32 * p + 16, 16)])
            for p in range(3)
        ]
        zvec = aux_v[pl.ds(96, 16)]

        def fire(k, buf, sem):
            pltpu.async_copy(
                pts_hbm.at[b, pl.ds(row0 + k * ROWS, ROWS), :],
                vbuf.at[buf],
                sem,
            )

        def wait(buf, sem):
            pltpu.make_async_copy(
                pts_hbm.at[b, pl.ds(row0, ROWS), :], vbuf.at[buf], sem
            ).wait()

        sems = (sem0, sem1, sem2, sem3)
        for j in range(NBUF - 1):
            fire(j, j, sems[j])

        def chunk_body(k, carry):
            par = lax.rem(k, NBUF)
            nxt = k + NBUF - 1
            nxt_par = lax.rem(nxt, NBUF)

            for j in range(NBUF):
                @pl.when(jnp.logical_and(nxt < n_chunks, nxt_par == j))
                def _(j=j):
                    fire(nxt, j, sems[j])

            for j in range(NBUF):
                @pl.when(par == j)
                def _(j=j):
                    wait(j, sems[j])

            sup = k // WRITE_EVERY
            ob = lax.rem(sup, 2)
            semos = (semo0, semo1)

            for j in range(2):
                @pl.when(
                    jnp.logical_and(
                        lax.rem(k, WRITE_EVERY) == 0,
                        jnp.logical_and(sup >= 2, ob == j),
                    )
                )
                def _(j=j):
                    pltpu.make_async_copy(
                        out_v.at[pl.ds(0, span)],
                        out_hbm.at[pl.ds(obase, span)],
                        semos[j],
                    ).wait()

            bvec = zvec + par
            vbase = ob * span + lax.rem(k, WRITE_EVERY) * (ROWS * C)

            for m in range(MACROS):
                for p, (iv, jv) in enumerate(phases):
                    vals = plsc.load_gather(vbuf, [bvec, iv + 8 * m, jv])
                    out_v[pl.ds(vbase + m * 48 + p * 16, 16)] = vals

            for j in range(2):
                @pl.when(
                    jnp.logical_and(
                        lax.rem(k, WRITE_EVERY) == WRITE_EVERY - 1, ob == j
                    )
                )
                def _(j=j):
                    pltpu.async_copy(
                        out_v.at[pl.ds(j * span, span)],
                        out_hbm.at[pl.ds(obase + sup * span, span)],
                        semos[j],
                    )

            return carry

        lax.fori_loop(0, n_chunks, chunk_body, 0)
        pltpu.make_async_copy(
            out_v.at[pl.ds(0, span)], out_hbm.at[pl.ds(obase, span)], semo0
        ).wait()
        pltpu.make_async_copy(
            out_v.at[pl.ds(0, span)], out_hbm.at[pl.ds(obase, span)], semo1
        ).wait()

    return flatten_kernel


@functools.lru_cache(maxsize=None)
def _build_gather(E: int):
    assert E % (_NUM_WORKERS * _CHUNK) == 0
    per_w = E // _NUM_WORKERS
    n_chunks = per_w // _CHUNK
    mesh = plsc.VectorSubcoreMesh(core_axis_name="c", subcore_axis_name="s")

    @functools.partial(
        pl.kernel,
        out_type=jax.ShapeDtypeStruct((E,), jnp.float32),
        mesh=mesh,
        scratch_types=[
            pltpu.VMEM((per_w,), jnp.int32),
            pltpu.VMEM((per_w,), jnp.float32),
            pltpu.SemaphoreType.DMA,
        ],
    )
    def gather_kernel(table_hbm, idx_hbm, out_hbm, idx_v, vals_v, sem):
        wid = lax.axis_index("s") * _NUM_CORES + lax.axis_index("c")
        base = wid * per_w
        pltpu.sync_copy(idx_hbm.at[pl.ds(base, per_w)], idx_v)

        def issue(c, carry):
            off = c * _CHUNK
            pltpu.async_copy(
                table_hbm.at[idx_v.at[pl.ds(off, _CHUNK)]],
                vals_v.at[pl.ds(off, _CHUNK)],
                sem,
            )
            return carry

        lax.fori_loop(0, n_chunks, issue, 0)
        pltpu.make_async_copy(table_hbm.at[pl.ds(0, per_w)], vals_v, sem).wait()
        pltpu.sync_copy(vals_v, out_hbm.at[pl.ds(base, per_w)])

    return gather_kernel


@functools.lru_cache(maxsize=None)
def _flatten_aux(C: int) -> np.ndarray:
    q = np.arange(48, dtype=np.int32)
    iv, jv = q // C, q % C
    parts = []
    for p in range(3):
        parts += [iv[16 * p : 16 * (p + 1)], jv[16 * p : 16 * (p + 1)]]
    parts.append(np.zeros(16, np.int32))
    return np.concatenate(parts)


def kernel(points):
    B, N, C = points.shape
    flat_idx = jnp.asarray(_flat_sample_indices(B, N, C))
    table = _build_flatten(B, N, C)(points, jnp.asarray(_flatten_aux(C)))
    out = _build_gather(B * _NUM_SAMPLE * C)(table, flat_idx)
    return out.reshape(B, _NUM_SAMPLE, C)
